# Initial kernel scaffold; baseline (speedup 1.0000x reference)
import jax
import jax.numpy as jnp
from jax.experimental import pallas as pl


def kernel(conv1_w, bn1_g, bn1_b, L1_B0_w1, L1_B0_g1, L1_B0_b1, L1_B0_w2, L1_B0_g2, L1_B0_b2, L1_B0_w3, L1_B0_g3, L1_B0_b3, L1_B0_wd, L1_B0_gd, L1_B0_bd, L1_B1_w1, L1_B1_g1, L1_B1_b1, L1_B1_w2, L1_B1_g2, L1_B1_b2, L1_B1_w3, L1_B1_g3, L1_B1_b3, L1_B2_w1, L1_B2_g1, L1_B2_b1, L1_B2_w2, L1_B2_g2, L1_B2_b2, L1_B2_w3, L1_B2_g3, L1_B2_b3, L2_B0_w1, L2_B0_g1, L2_B0_b1, L2_B0_w2, L2_B0_g2, L2_B0_b2, L2_B0_w3, L2_B0_g3, L2_B0_b3, L2_B0_wd, L2_B0_gd, L2_B0_bd, L2_B1_w1, L2_B1_g1, L2_B1_b1, L2_B1_w2, L2_B1_g2, L2_B1_b2, L2_B1_w3, L2_B1_g3, L2_B1_b3, L2_B2_w1, L2_B2_g1, L2_B2_b1, L2_B2_w2, L2_B2_g2, L2_B2_b2, L2_B2_w3, L2_B2_g3, L2_B2_b3, L2_B3_w1, L2_B3_g1, L2_B3_b1, L2_B3_w2, L2_B3_g2, L2_B3_b2, L2_B3_w3, L2_B3_g3, L2_B3_b3, L3_B0_w1, L3_B0_g1, L3_B0_b1, L3_B0_w2, L3_B0_g2, L3_B0_b2, L3_B0_w3, L3_B0_g3, L3_B0_b3, L3_B0_wd, L3_B0_gd, L3_B0_bd, L3_B1_w1, L3_B1_g1, L3_B1_b1, L3_B1_w2, L3_B1_g2, L3_B1_b2, L3_B1_w3, L3_B1_g3, L3_B1_b3, L3_B2_w1, L3_B2_g1, L3_B2_b1, L3_B2_w2, L3_B2_g2, L3_B2_b2, L3_B2_w3, L3_B2_g3, L3_B2_b3, L3_B3_w1, L3_B3_g1, L3_B3_b1, L3_B3_w2, L3_B3_g2, L3_B3_b2, L3_B3_w3, L3_B3_g3, L3_B3_b3, L3_B4_w1, L3_B4_g1, L3_B4_b1, L3_B4_w2, L3_B4_g2, L3_B4_b2, L3_B4_w3, L3_B4_g3, L3_B4_b3, L3_B5_w1, L3_B5_g1, L3_B5_b1, L3_B5_w2, L3_B5_g2, L3_B5_b2, L3_B5_w3, L3_B5_g3, L3_B5_b3, L4_B0_w1, L4_B0_g1, L4_B0_b1, L4_B0_w2, L4_B0_g2, L4_B0_b2, L4_B0_w3, L4_B0_g3, L4_B0_b3, L4_B0_wd, L4_B0_gd, L4_B0_bd, L4_B1_w1, L4_B1_g1, L4_B1_b1, L4_B1_w2, L4_B1_g2, L4_B1_b2, L4_B1_w3, L4_B1_g3, L4_B1_b3, L4_B2_w1, L4_B2_g1, L4_B2_b1, L4_B2_w2, L4_B2_g2, L4_B2_b2, L4_B2_w3, L4_B2_g3, L4_B2_b3, x):
    raise NotImplementedError("write your pallas kernel here")


# speedup vs baseline: 1.3291x; 1.3291x over previous
import functools

import jax
import jax.numpy as jnp
from jax.experimental import pallas as pl
from jax.experimental.pallas import tpu as pltpu

_EPS = 1e-5
_ROW = 256
_BIG = 1e6


def _round_up(x, m):
    return (x + m - 1) // m * m


def _pick_tile(dim, max_tile, align):
    padded = _round_up(dim, align)
    tile = align
    t = align
    limit = min(max_tile, padded)
    while t <= limit:
        if padded % t == 0:
            tile = t
        t += align
    return tile, padded


def _mm_kernel(a_ref, b_ref, o_ref, acc_ref):
    @pl.when(pl.program_id(2) == 0)
    def _():
        acc_ref[...] = jnp.zeros_like(acc_ref)

    acc_ref[...] += jnp.dot(a_ref[...], b_ref[...],
                            preferred_element_type=jnp.float32)

    @pl.when(pl.program_id(2) == pl.num_programs(2) - 1)
    def _():
        o_ref[...] = acc_ref[...].astype(o_ref.dtype)


def _mm_norm_kernel(a_ref, s_ref, ss_ref, g_ref, be_ref, b_ref, o_ref,
                    acc_ref, *, inv_m):
    @pl.when(pl.program_id(2) == 0)
    def _():
        acc_ref[...] = jnp.zeros_like(acc_ref)

    mean = s_ref[...] * inv_m
    var = jnp.maximum(ss_ref[...] * inv_m - mean * mean, 0.0)
    scale = jax.lax.rsqrt(var + _EPS) * g_ref[...]
    a = a_ref[...].astype(jnp.float32)
    a = jnp.maximum((a - mean) * scale + be_ref[...], 0.0).astype(jnp.bfloat16)
    acc_ref[...] += jnp.dot(a, b_ref[...],
                            preferred_element_type=jnp.float32)

    @pl.when(pl.program_id(2) == pl.num_programs(2) - 1)
    def _():
        o_ref[...] = acc_ref[...].astype(o_ref.dtype)


def _matmul(a, b, norm=None, out_dtype=jnp.bfloat16):
    M, K = a.shape
    K2, N = b.shape
    assert K == K2, (K, K2)
    TM, Mp = _pick_tile(M, 256, 16)
    TK, Kp = _pick_tile(K, 512, 128)
    TN, Np = _pick_tile(N, 512, 128)

    a = a.astype(jnp.bfloat16)
    b = b.astype(jnp.bfloat16)
    if (Mp, Kp) != (M, K):
        a = jnp.pad(a, ((0, Mp - M), (0, Kp - K)))
    if (Kp, Np) != (K, N):
        b = jnp.pad(b, ((0, Kp - K), (0, Np - N)))

    a_spec = pl.BlockSpec((TM, TK), lambda i, j, k: (i, k))
    b_spec = pl.BlockSpec((TK, TN), lambda i, j, k: (k, j))
    if norm is None:
        kern = _mm_kernel
        ins = (a, b)
        in_specs = [a_spec, b_spec]
    else:
        s, ss, g, be, inv_m = norm
        if Kp != K:
            pad = ((0, 0), (0, Kp - K))
            s, ss, g, be = (jnp.pad(v.astype(jnp.float32), pad)
                            for v in (s, ss, g, be))
        else:
            s, ss, g, be = (v.astype(jnp.float32) for v in (s, ss, g, be))
        kern = functools.partial(_mm_norm_kernel, inv_m=inv_m)
        vec_spec = pl.BlockSpec((1, TK), lambda i, j, k: (0, k))
        ins = (a, s, ss, g, be, b)
        in_specs = [a_spec, vec_spec, vec_spec, vec_spec, vec_spec, b_spec]

    out = pl.pallas_call(
        kern,
        out_shape=jax.ShapeDtypeStruct((Mp, Np), out_dtype),
        grid_spec=pltpu.PrefetchScalarGridSpec(
            num_scalar_prefetch=0,
            grid=(Mp // TM, Np // TN, Kp // TK),
            in_specs=in_specs,
            out_specs=pl.BlockSpec((TM, TN), lambda i, j, k: (i, j)),
            scratch_shapes=[pltpu.VMEM((TM, TN), jnp.float32)],
        ),
        compiler_params=pltpu.CompilerParams(
            dimension_semantics=("parallel", "parallel", "arbitrary")),
    )(*ins)

    if (Mp, Np) != (M, N):
        out = out[:M, :N]
    return out


def _stats_kernel(x_ref, sum_ref, ssq_ref, sum_acc, ssq_acc):
    @pl.when(pl.program_id(0) == 0)
    def _():
        sum_acc[...] = jnp.zeros_like(sum_acc)
        ssq_acc[...] = jnp.zeros_like(ssq_acc)

    x = x_ref[...].astype(jnp.float32)
    sum_acc[...] += jnp.sum(x, axis=0, keepdims=True)
    ssq_acc[...] += jnp.sum(x * x, axis=0, keepdims=True)

    @pl.when(pl.program_id(0) == pl.num_programs(0) - 1)
    def _():
        sum_ref[...] = sum_acc[...]
        ssq_ref[...] = ssq_acc[...]


def _bn_stats(x2):
    M, C = x2.shape
    TMR, Mp = _pick_tile(M, _ROW, 16)
    if Mp != M:
        x2 = jnp.pad(x2, ((0, Mp - M), (0, 0)))
    return pl.pallas_call(
        _stats_kernel,
        out_shape=(jax.ShapeDtypeStruct((1, C), jnp.float32),
                   jax.ShapeDtypeStruct((1, C), jnp.float32)),
        grid_spec=pltpu.PrefetchScalarGridSpec(
            num_scalar_prefetch=0,
            grid=(Mp // TMR,),
            in_specs=[pl.BlockSpec((TMR, C), lambda i: (i, 0))],
            out_specs=(pl.BlockSpec((1, C), lambda i: (0, 0)),
                       pl.BlockSpec((1, C), lambda i: (0, 0))),
            scratch_shapes=[pltpu.VMEM((1, C), jnp.float32),
                            pltpu.VMEM((1, C), jnp.float32)],
        ),
        compiler_params=pltpu.CompilerParams(
            dimension_semantics=("arbitrary",)),
    )(x2)


def _pad_hw(xn, H, W, C, value):
    zrow = jnp.full((1, W + 2, C), value, jnp.bfloat16)
    zcol = jnp.full((H, 1, C), value, jnp.bfloat16)
    mid = jnp.concatenate([zcol, xn, zcol], axis=1)
    return jnp.concatenate([zrow, mid, zrow], axis=0)


def _subsample2(sl, Ho, Wo, C):
    sl = sl.reshape(Ho, 2, 2 * Wo, C)[:, 0]
    return sl.reshape(Ho, Wo, 2, C)[:, :, 0]


def _pool_kernel(a_ref, s_ref, ss_ref, g_ref, be_ref, o_ref, *, H, W, C, inv_m):
    mean = s_ref[...] * inv_m
    var = jnp.maximum(ss_ref[...] * inv_m - mean * mean, 0.0)
    scale = jax.lax.rsqrt(var + _EPS) * g_ref[...]
    z = (a_ref[0].astype(jnp.float32) - mean) * scale + be_ref[...]
    zb = jnp.maximum(z, 0.0).astype(jnp.bfloat16)
    xp = _pad_hw(zb, H, W, C, 0)
    Ho, Wo = H // 2, W // 2
    m = None
    for di in range(3):
        for dj in range(3):
            sl = _subsample2(xp[di:di + 2 * Ho, dj:dj + 2 * Wo, :], Ho, Wo, C)
            m = sl if m is None else jnp.maximum(m, sl)
    o_ref[0] = m


def _pool_bn(y, s, ss, g, be, inv_m):
    N, H, W, C = y.shape
    Ho, Wo = H // 2, W // 2
    kern = functools.partial(_pool_kernel, H=H, W=W, C=C, inv_m=inv_m)
    vec = pl.BlockSpec((1, C), lambda i: (0, 0))
    return pl.pallas_call(
        kern,
        out_shape=jax.ShapeDtypeStruct((N, Ho, Wo, C), jnp.bfloat16),
        grid=(N,),
        in_specs=[pl.BlockSpec((1, H, W, C), lambda i: (i, 0, 0, 0)),
                  vec, vec, vec, vec],
        out_specs=pl.BlockSpec((1, Ho, Wo, C), lambda i: (i, 0, 0, 0)),
        compiler_params=pltpu.CompilerParams(
            dimension_semantics=("parallel",)),
    )(y, s, ss, g.astype(jnp.float32).reshape(1, C),
      be.astype(jnp.float32).reshape(1, C))


def _bn_vec(s_ref, ss_ref, g_ref, inv_m):
    mean = s_ref[...] * inv_m
    var = jnp.maximum(ss_ref[...] * inv_m - mean * mean, 0.0)
    return mean, jax.lax.rsqrt(var + _EPS) * g_ref[...]


def _tail_id_kernel(y_ref, s_ref, ss_ref, g_ref, be_ref, r_ref, o_ref, *, inv_m):
    mean, scale = _bn_vec(s_ref, ss_ref, g_ref, inv_m)
    y = (y_ref[...].astype(jnp.float32) - mean) * scale + be_ref[...]
    y = y + r_ref[...].astype(jnp.float32)
    o_ref[...] = jnp.maximum(y, 0.0).astype(o_ref.dtype)


def _tail_ds_kernel(y_ref, s_ref, ss_ref, g_ref, be_ref,
                    yd_ref, sd_ref, ssd_ref, gd_ref, bed_ref, o_ref, *, inv_m):
    mean, scale = _bn_vec(s_ref, ss_ref, g_ref, inv_m)
    meand, scaled = _bn_vec(sd_ref, ssd_ref, gd_ref, inv_m)
    r = (yd_ref[...].astype(jnp.float32) - meand) * scaled + bed_ref[...]
    y = (y_ref[...].astype(jnp.float32) - mean) * scale + be_ref[...]
    y = y + r.astype(jnp.bfloat16).astype(jnp.float32)
    o_ref[...] = jnp.maximum(y, 0.0).astype(o_ref.dtype)


def _block_tail(y3, bn3, res, ds=None):
    M, C = y3.shape
    s3, ss3, g3, be3, inv_m = bn3
    TMR, Mp = _pick_tile(M, _ROW, 16)
    if Mp != M:
        y3 = jnp.pad(y3, ((0, Mp - M), (0, 0)))
        res = jnp.pad(res, ((0, Mp - M), (0, 0)))
    row = pl.BlockSpec((TMR, C), lambda i: (i, 0))
    vec = pl.BlockSpec((1, C), lambda i: (0, 0))
    f32 = jnp.float32
    if ds is None:
        kern = functools.partial(_tail_id_kernel, inv_m=inv_m)
        ins = (y3, s3, ss3, g3.astype(f32).reshape(1, C),
               be3.astype(f32).reshape(1, C), res)
        in_specs = [row, vec, vec, vec, vec, row]
    else:
        sd, ssd, gd, bed = ds
        kern = functools.partial(_tail_ds_kernel, inv_m=inv_m)
        ins = (y3, s3, ss3, g3.astype(f32).reshape(1, C),
               be3.astype(f32).reshape(1, C),
               res, sd, ssd, gd.astype(f32).reshape(1, C),
               bed.astype(f32).reshape(1, C))
        in_specs = [row, vec, vec, vec, vec, row, vec, vec, vec, vec]
    out = pl.pallas_call(
        kern,
        out_shape=jax.ShapeDtypeStruct((Mp, C), jnp.bfloat16),
        grid_spec=pltpu.PrefetchScalarGridSpec(
            num_scalar_prefetch=0,
            grid=(Mp // TMR,),
            in_specs=in_specs,
            out_specs=row,
        ),
        compiler_params=pltpu.CompilerParams(
            dimension_semantics=("parallel",)),
    )(*ins)
    return out[:M] if Mp != M else out


def _extract(x, kh, kw, stride, padding):
    N, H, W, C = x.shape
    if padding > 0:
        x = jnp.pad(x, ((0, 0), (padding, padding), (padding, padding), (0, 0)))
    Hp, Wp = H + 2 * padding, W + 2 * padding
    Ho = (Hp - kh) // stride + 1
    Wo = (Wp - kw) // stride + 1
    cols = []
    for i in range(kh):
        for j in range(kw):
            cols.append(x[:, i: i + (Ho - 1) * stride + 1: stride,
                          j: j + (Wo - 1) * stride + 1: stride, :])
    return jnp.concatenate(cols, axis=-1).reshape(N * Ho * Wo, kh * kw * C), Ho, Wo


def _border(x4, v):
    N, H, W, C = x4.shape
    vb = v.astype(jnp.bfloat16).reshape(1, 1, 1, C)
    col = jnp.broadcast_to(vb, (N, H, 1, C))
    x4 = jnp.concatenate([col, x4, col], axis=2)
    row = jnp.broadcast_to(vb, (N, 1, W + 2, C))
    return jnp.concatenate([row, x4, row], axis=1)


def _conv3x3_fused(y1, bn1, w, stride):
    N, H, W, C = y1.shape
    s, ss, g, be, inv_m = bn1
    mean = s * inv_m
    var = jnp.maximum(ss * inv_m - mean * mean, 0.0)
    scale = jax.lax.rsqrt(var + _EPS) * g.astype(jnp.float32).reshape(1, C)
    beta = be.astype(jnp.float32).reshape(1, C)
    v = (mean - (beta + _BIG) / scale).reshape(C)
    patches, Ho, Wo = _extract(_border(y1, v), 3, 3, stride, 0)
    t9 = lambda u: jnp.tile(u.reshape(1, C), (1, 9))
    y2 = _matmul(patches, w, norm=(t9(s), t9(ss),
                                   t9(g.astype(jnp.float32)),
                                   t9(be.astype(jnp.float32)), inv_m))
    return y2, Ho, Wo


_CFG = (((1, True), (1, False), (1, False)),
        ((2, True), (1, False), (1, False), (1, False)),
        ((2, True), (1, False), (1, False), (1, False), (1, False), (1, False)),
        ((2, True), (1, False), (1, False)))


def kernel(conv1_w, bn1_g, bn1_b, L1_B0_w1, L1_B0_g1, L1_B0_b1, L1_B0_w2, L1_B0_g2, L1_B0_b2, L1_B0_w3, L1_B0_g3, L1_B0_b3, L1_B0_wd, L1_B0_gd, L1_B0_bd, L1_B1_w1, L1_B1_g1, L1_B1_b1, L1_B1_w2, L1_B1_g2, L1_B1_b2, L1_B1_w3, L1_B1_g3, L1_B1_b3, L1_B2_w1, L1_B2_g1, L1_B2_b1, L1_B2_w2, L1_B2_g2, L1_B2_b2, L1_B2_w3, L1_B2_g3, L1_B2_b3, L2_B0_w1, L2_B0_g1, L2_B0_b1, L2_B0_w2, L2_B0_g2, L2_B0_b2, L2_B0_w3, L2_B0_g3, L2_B0_b3, L2_B0_wd, L2_B0_gd, L2_B0_bd, L2_B1_w1, L2_B1_g1, L2_B1_b1, L2_B1_w2, L2_B1_g2, L2_B1_b2, L2_B1_w3, L2_B1_g3, L2_B1_b3, L2_B2_w1, L2_B2_g1, L2_B2_b1, L2_B2_w2, L2_B2_g2, L2_B2_b2, L2_B2_w3, L2_B2_g3, L2_B2_b3, L2_B3_w1, L2_B3_g1, L2_B3_b1, L2_B3_w2, L2_B3_g2, L2_B3_b2, L2_B3_w3, L2_B3_g3, L2_B3_b3, L3_B0_w1, L3_B0_g1, L3_B0_b1, L3_B0_w2, L3_B0_g2, L3_B0_b2, L3_B0_w3, L3_B0_g3, L3_B0_b3, L3_B0_wd, L3_B0_gd, L3_B0_bd, L3_B1_w1, L3_B1_g1, L3_B1_b1, L3_B1_w2, L3_B1_g2, L3_B1_b2, L3_B1_w3, L3_B1_g3, L3_B1_b3, L3_B2_w1, L3_B2_g1, L3_B2_b1, L3_B2_w2, L3_B2_g2, L3_B2_b2, L3_B2_w3, L3_B2_g3, L3_B2_b3, L3_B3_w1, L3_B3_g1, L3_B3_b1, L3_B3_w2, L3_B3_g2, L3_B3_b2, L3_B3_w3, L3_B3_g3, L3_B3_b3, L3_B4_w1, L3_B4_g1, L3_B4_b1, L3_B4_w2, L3_B4_g2, L3_B4_b2, L3_B4_w3, L3_B4_g3, L3_B4_b3, L3_B5_w1, L3_B5_g1, L3_B5_b1, L3_B5_w2, L3_B5_g2, L3_B5_b2, L3_B5_w3, L3_B5_g3, L3_B5_b3, L4_B0_w1, L4_B0_g1, L4_B0_b1, L4_B0_w2, L4_B0_g2, L4_B0_b2, L4_B0_w3, L4_B0_g3, L4_B0_b3, L4_B0_wd, L4_B0_gd, L4_B0_bd, L4_B1_w1, L4_B1_g1, L4_B1_b1, L4_B1_w2, L4_B1_g2, L4_B1_b2, L4_B1_w3, L4_B1_g3, L4_B1_b3, L4_B2_w1, L4_B2_g1, L4_B2_b1, L4_B2_w2, L4_B2_g2, L4_B2_b2, L4_B2_w3, L4_B2_g3, L4_B2_b3, x):
    d = dict(locals())

    xh = jnp.transpose(x, (0, 2, 3, 1)).astype(jnp.bfloat16)
    N = xh.shape[0]
    patches, Ho, Wo = _extract(xh, 7, 7, 2, 3)
    y = _matmul(patches, conv1_w)
    s, ss = _bn_stats(y)
    cur = _pool_bn(y.reshape(N, Ho, Wo, 64), s, ss, bn1_g, bn1_b,
                   1.0 / float(N * Ho * Wo))

    low = None
    for li, layer_cfg in enumerate(_CFG):
        for bi, (st, has_ds) in enumerate(layer_cfg):
            p = lambda n: d['L%d_B%d_%s' % (li + 1, bi, n)]
            Nn, H, W, C = cur.shape
            Cp = p('w1').shape[1]
            x2 = cur.reshape(Nn * H * W, C)
            inv1 = 1.0 / float(Nn * H * W)

            y1 = _matmul(x2, p('w1'))
            s1, ss1 = _bn_stats(y1)
            y2, Ho2, Wo2 = _conv3x3_fused(
                y1.reshape(Nn, H, W, Cp), (s1, ss1, p('g1'), p('b1'), inv1),
                p('w2'), st)
            M2 = Nn * Ho2 * Wo2
            inv2 = 1.0 / float(M2)
            s2, ss2 = _bn_stats(y2)
            y3 = _matmul(y2, p('w3'),
                         norm=(s2, ss2,
                               p('g2').astype(jnp.float32).reshape(1, Cp),
                               p('b2').astype(jnp.float32).reshape(1, Cp),
                               inv2))
            s3, ss3 = _bn_stats(y3)
            bn3 = (s3, ss3, p('g3'), p('b3'), inv2)
            if has_ds:
                xin = cur if st == 1 else cur[:, ::2, ::2, :]
                yd = _matmul(xin.reshape(M2, C), p('wd'))
                sd, ssd = _bn_stats(yd)
                nxt = _block_tail(y3, bn3, yd, ds=(sd, ssd, p('gd'), p('bd')))
            else:
                nxt = _block_tail(y3, bn3, x2)
            cur = nxt.reshape(Nn, Ho2, Wo2, 4 * Cp)
        if li == 0:
            low = cur

    return (jnp.transpose(cur, (0, 3, 1, 2)).astype(jnp.float32),
            jnp.transpose(low, (0, 3, 1, 2)).astype(jnp.float32))
